# baseline (device time: 645832 ns/iter reference)
import functools

import jax
import jax.numpy as jnp
from jax import lax
from jax.experimental import pallas as pl
from jax.experimental.pallas import tpu as pltpu

N_DEV = 16
B_PER = 2
SQ = 128
SKV = 128
D = 512
H_PER = 8
DH = 64
SCALE = 0.125


def _ag_body(x_ref, out_ref, comm_ref, send_sems, recv_sems):
    my = lax.axis_index("i")
    left = (my - 1) % N_DEV
    right = (my + 1) % N_DEV

    barrier_sem = pltpu.get_barrier_semaphore()
    for nbr in [left, right]:
        pl.semaphore_signal(
            barrier_sem, inc=1,
            device_id=(nbr,), device_id_type=pl.DeviceIdType.MESH,
        )
    pl.semaphore_wait(barrier_sem, 2)

    mine = x_ref[...].astype(jnp.bfloat16)
    out_ref[pl.ds(my * B_PER, B_PER), :, :] = mine
    comm_ref[0] = mine

    for h in range(N_DEV - 1):
        send_slot = h % 2
        recv_slot = (h + 1) % 2
        rdma = pltpu.make_async_remote_copy(
            src_ref=comm_ref.at[send_slot],
            dst_ref=comm_ref.at[recv_slot],
            send_sem=send_sems.at[send_slot],
            recv_sem=recv_sems.at[recv_slot],
            device_id=(right,),
            device_id_type=pl.DeviceIdType.MESH,
        )
        rdma.start()
        rdma.wait()
        origin = (my - h - 1) % N_DEV
        out_ref[pl.ds(origin * B_PER, B_PER), :, :] = comm_ref[recv_slot]


def _all_gather_x(x):
    return pl.pallas_call(
        _ag_body,
        out_shape=jax.ShapeDtypeStruct((N_DEV * B_PER, SQ, D), jnp.bfloat16),
        in_specs=[pl.BlockSpec(memory_space=pltpu.VMEM)],
        out_specs=pl.BlockSpec(memory_space=pltpu.VMEM),
        scratch_shapes=[
            pltpu.VMEM((2, B_PER, SQ, D), jnp.bfloat16),
            pltpu.SemaphoreType.DMA((2,)),
            pltpu.SemaphoreType.DMA((2,)),
        ],
        compiler_params=pltpu.CompilerParams(collective_id=0),
    )(x)


def _attn_body(idx_ref, x_ref, wq_ref, wo_ref, k_ref, v_ref, part_ref):
    del idx_ref
    wq = wq_ref[...].astype(jnp.bfloat16)
    wo = wo_ref[...].astype(jnp.bfloat16)
    q = jnp.dot(x_ref[0], wq, preferred_element_type=jnp.float32)
    outs = []
    for h in range(H_PER):
        q_h = q[:, h * DH:(h + 1) * DH].astype(jnp.bfloat16)
        k_h = k_ref[0, :, h, :].astype(jnp.bfloat16)
        v_h = v_ref[0, :, h, :].astype(jnp.bfloat16)
        s = lax.dot_general(
            q_h, k_h, (((1,), (1,)), ((), ())),
            preferred_element_type=jnp.float32,
        ) * SCALE
        m = jnp.max(s, axis=1, keepdims=True)
        p = jnp.exp(s - m)
        l = jnp.sum(p, axis=1, keepdims=True)
        pv = jnp.dot(p.astype(jnp.bfloat16), v_h,
                     preferred_element_type=jnp.float32)
        outs.append(pv / l)
    o = jnp.concatenate(outs, axis=1).astype(jnp.bfloat16)
    part_ref[0] = jnp.dot(o, wo, preferred_element_type=jnp.float32)


def _partials(x_all, Wq, Wo, K_ext, V_ext, head_idx):
    B = N_DEV * B_PER
    grid_spec = pltpu.PrefetchScalarGridSpec(
        num_scalar_prefetch=1,
        grid=(B,),
        in_specs=[
            pl.BlockSpec((1, SQ, D), lambda b, I: (b, 0, 0)),
            pl.BlockSpec((D, H_PER * DH), lambda b, I: (0, 0)),
            pl.BlockSpec((H_PER * DH, D), lambda b, I: (0, 0)),
            pl.BlockSpec((1, SKV, H_PER, DH), lambda b, I: (b, 0, I[0], 0)),
            pl.BlockSpec((1, SKV, H_PER, DH), lambda b, I: (b, 0, I[0], 0)),
        ],
        out_specs=pl.BlockSpec((1, SQ, D), lambda b, I: (b, 0, 0)),
    )
    return pl.pallas_call(
        _attn_body,
        grid_spec=grid_spec,
        out_shape=jax.ShapeDtypeStruct((B, SQ, D), jnp.float32),
    )(head_idx, x_all, Wq, Wo, K_ext, V_ext)


def _rs_body(part_ref, out_ref, sbuf, rbuf, send_sems, recv_sems):
    my = lax.axis_index("i")
    left = (my - 1) % N_DEV
    right = (my + 1) % N_DEV

    barrier_sem = pltpu.get_barrier_semaphore()
    for nbr in [left, right]:
        pl.semaphore_signal(
            barrier_sem, inc=1,
            device_id=(nbr,), device_id_type=pl.DeviceIdType.MESH,
        )
    pl.semaphore_wait(barrier_sem, 2)

    for s in range(N_DEV - 1):
        c = (my - s - 1) % N_DEV
        val = part_ref[pl.ds(c * B_PER, B_PER), :, :]
        if s > 0:
            val = val + rbuf[(s - 1) % 2]
        sbuf[s % 2] = val
        rdma = pltpu.make_async_remote_copy(
            src_ref=sbuf.at[s % 2],
            dst_ref=rbuf.at[s % 2],
            send_sem=send_sems.at[s % 2],
            recv_sem=recv_sems.at[s % 2],
            device_id=(right,),
            device_id_type=pl.DeviceIdType.MESH,
        )
        rdma.start()
        rdma.wait()

    out_ref[...] = part_ref[pl.ds(my * B_PER, B_PER), :, :] + rbuf[(N_DEV - 2) % 2]


def _reduce_scatter(partial):
    return pl.pallas_call(
        _rs_body,
        out_shape=jax.ShapeDtypeStruct((B_PER, SQ, D), jnp.float32),
        in_specs=[pl.BlockSpec(memory_space=pltpu.VMEM)],
        out_specs=pl.BlockSpec(memory_space=pltpu.VMEM),
        scratch_shapes=[
            pltpu.VMEM((2, B_PER, SQ, D), jnp.float32),
            pltpu.VMEM((2, B_PER, SQ, D), jnp.float32),
            pltpu.SemaphoreType.DMA((2,)),
            pltpu.SemaphoreType.DMA((2,)),
        ],
        compiler_params=pltpu.CompilerParams(collective_id=1),
    )(partial)


def kernel(x, Wq, Wo, K_ext, V_ext):
    head_idx = jnp.full((1,), lax.axis_index("i"), jnp.int32)
    x_all = _all_gather_x(x)
    partial = _partials(x_all, Wq, Wo, K_ext, V_ext, head_idx)
    return _reduce_scatter(partial)


# device time: 604688 ns/iter; 1.0680x vs baseline; 1.0680x over previous
import functools

import jax
import jax.numpy as jnp
from jax import lax
from jax.experimental import pallas as pl
from jax.experimental.pallas import tpu as pltpu

N_DEV = 16
B_PER = 2
SQ = 128
SKV = 128
D = 512
H_PER = 8
DH = 64
SCALE = 0.125


def _ag_body(x_ref, out_ref, comm_ref, send_sems, recv_sems):
    my = lax.axis_index("i")
    left = (my - 1) % N_DEV
    right = (my + 1) % N_DEV

    barrier_sem = pltpu.get_barrier_semaphore()
    for nbr in [left, right]:
        pl.semaphore_signal(
            barrier_sem, inc=1,
            device_id=(nbr,), device_id_type=pl.DeviceIdType.MESH,
        )
    pl.semaphore_wait(barrier_sem, 2)

    mine = x_ref[...].astype(jnp.bfloat16)
    out_ref[pl.ds(my * B_PER, B_PER), :, :] = mine
    comm_ref[0] = mine

    for h in range(N_DEV - 1):
        send_slot = h % 2
        recv_slot = (h + 1) % 2
        rdma = pltpu.make_async_remote_copy(
            src_ref=comm_ref.at[send_slot],
            dst_ref=comm_ref.at[recv_slot],
            send_sem=send_sems.at[send_slot],
            recv_sem=recv_sems.at[recv_slot],
            device_id=(right,),
            device_id_type=pl.DeviceIdType.MESH,
        )
        rdma.start()
        rdma.wait()
        origin = (my - h - 1) % N_DEV
        out_ref[pl.ds(origin * B_PER, B_PER), :, :] = comm_ref[recv_slot]


def _all_gather_x(x):
    return pl.pallas_call(
        _ag_body,
        out_shape=jax.ShapeDtypeStruct((N_DEV * B_PER, SQ, D), jnp.bfloat16),
        in_specs=[pl.BlockSpec(memory_space=pltpu.VMEM)],
        out_specs=pl.BlockSpec(memory_space=pltpu.VMEM),
        scratch_shapes=[
            pltpu.VMEM((2, B_PER, SQ, D), jnp.bfloat16),
            pltpu.SemaphoreType.DMA((2,)),
            pltpu.SemaphoreType.DMA((2,)),
        ],
        compiler_params=pltpu.CompilerParams(collective_id=0),
    )(x)


def _attn_body(x_ref, wq_ref, wo_ref, k_ref, v_ref, part_ref):
    wq = wq_ref[...].astype(jnp.bfloat16)
    wo = wo_ref[...].astype(jnp.bfloat16)
    q = jnp.dot(x_ref[0], wq, preferred_element_type=jnp.float32)
    outs = []
    for h in range(H_PER):
        q_h = q[:, h * DH:(h + 1) * DH].astype(jnp.bfloat16)
        k_h = k_ref[0, :, h * DH:(h + 1) * DH].astype(jnp.bfloat16)
        v_h = v_ref[0, :, h * DH:(h + 1) * DH].astype(jnp.bfloat16)
        s = lax.dot_general(
            q_h, k_h, (((1,), (1,)), ((), ())),
            preferred_element_type=jnp.float32,
        ) * SCALE
        m = jnp.max(s, axis=1, keepdims=True)
        p = jnp.exp(s - m)
        l = jnp.sum(p, axis=1, keepdims=True)
        pv = jnp.dot(p.astype(jnp.bfloat16), v_h,
                     preferred_element_type=jnp.float32)
        outs.append(pv / l)
    o = jnp.concatenate(outs, axis=1).astype(jnp.bfloat16)
    part_ref[0] = jnp.dot(o, wo, preferred_element_type=jnp.float32)


def _partials(x_all, Wq, Wo, k_loc, v_loc):
    B = N_DEV * B_PER
    return pl.pallas_call(
        _attn_body,
        grid=(B,),
        in_specs=[
            pl.BlockSpec((1, SQ, D), lambda b: (b, 0, 0)),
            pl.BlockSpec((D, H_PER * DH), lambda b: (0, 0)),
            pl.BlockSpec((H_PER * DH, D), lambda b: (0, 0)),
            pl.BlockSpec((1, SKV, H_PER * DH), lambda b: (b, 0, 0)),
            pl.BlockSpec((1, SKV, H_PER * DH), lambda b: (b, 0, 0)),
        ],
        out_specs=pl.BlockSpec((1, SQ, D), lambda b: (b, 0, 0)),
        out_shape=jax.ShapeDtypeStruct((B, SQ, D), jnp.float32),
    )(x_all, Wq, Wo, k_loc, v_loc)


def _rs_body(part_ref, out_ref, sbuf, rbuf, send_sems, recv_sems):
    my = lax.axis_index("i")
    left = (my - 1) % N_DEV
    right = (my + 1) % N_DEV

    barrier_sem = pltpu.get_barrier_semaphore()
    for nbr in [left, right]:
        pl.semaphore_signal(
            barrier_sem, inc=1,
            device_id=(nbr,), device_id_type=pl.DeviceIdType.MESH,
        )
    pl.semaphore_wait(barrier_sem, 2)

    for s in range(N_DEV - 1):
        c = (my - s - 1) % N_DEV
        val = part_ref[pl.ds(c * B_PER, B_PER), :, :]
        if s > 0:
            val = val + rbuf[(s - 1) % 2]
        sbuf[s % 2] = val
        rdma = pltpu.make_async_remote_copy(
            src_ref=sbuf.at[s % 2],
            dst_ref=rbuf.at[s % 2],
            send_sem=send_sems.at[s % 2],
            recv_sem=recv_sems.at[s % 2],
            device_id=(right,),
            device_id_type=pl.DeviceIdType.MESH,
        )
        rdma.start()
        rdma.wait()

    out_ref[...] = part_ref[pl.ds(my * B_PER, B_PER), :, :] + rbuf[(N_DEV - 2) % 2]


def _reduce_scatter(partial):
    return pl.pallas_call(
        _rs_body,
        out_shape=jax.ShapeDtypeStruct((B_PER, SQ, D), jnp.float32),
        in_specs=[pl.BlockSpec(memory_space=pltpu.VMEM)],
        out_specs=pl.BlockSpec(memory_space=pltpu.VMEM),
        scratch_shapes=[
            pltpu.VMEM((2, B_PER, SQ, D), jnp.float32),
            pltpu.VMEM((2, B_PER, SQ, D), jnp.float32),
            pltpu.SemaphoreType.DMA((2,)),
            pltpu.SemaphoreType.DMA((2,)),
        ],
        compiler_params=pltpu.CompilerParams(collective_id=1),
    )(partial)


def kernel(x, Wq, Wo, K_ext, V_ext):
    i = lax.axis_index("i")
    B = N_DEV * B_PER
    k_loc = lax.dynamic_slice_in_dim(K_ext, i * H_PER, H_PER, axis=2)
    v_loc = lax.dynamic_slice_in_dim(V_ext, i * H_PER, H_PER, axis=2)
    k_loc = k_loc.reshape(B, SKV, H_PER * DH)
    v_loc = v_loc.reshape(B, SKV, H_PER * DH)
    x_all = _all_gather_x(x)
    partial = _partials(x_all, Wq, Wo, k_loc, v_loc)
    return _reduce_scatter(partial)


# device time: 526695 ns/iter; 1.2262x vs baseline; 1.1481x over previous
import functools

import jax
import jax.numpy as jnp
from jax import lax
from jax.experimental import pallas as pl
from jax.experimental.pallas import tpu as pltpu

N_DEV = 16
B_PER = 2
SQ = 128
SKV = 128
D = 512
H_PER = 8
DH = 64
SCALE = 0.125


def _ag_body(x_ref, out_ref, comm_ref, send_sems, recv_sems):
    my = lax.axis_index("i")
    left = (my - 1) % N_DEV
    right = (my + 1) % N_DEV

    barrier_sem = pltpu.get_barrier_semaphore()
    for nbr in [left, right]:
        pl.semaphore_signal(
            barrier_sem, inc=1,
            device_id=(nbr,), device_id_type=pl.DeviceIdType.MESH,
        )
    pl.semaphore_wait(barrier_sem, 2)

    mine = x_ref[...].astype(jnp.bfloat16)
    out_ref[pl.ds(my * B_PER, B_PER), :, :] = mine
    comm_ref[0] = mine

    for h in range(N_DEV - 1):
        send_slot = h % 2
        recv_slot = (h + 1) % 2
        rdma = pltpu.make_async_remote_copy(
            src_ref=comm_ref.at[send_slot],
            dst_ref=comm_ref.at[recv_slot],
            send_sem=send_sems.at[send_slot],
            recv_sem=recv_sems.at[recv_slot],
            device_id=(right,),
            device_id_type=pl.DeviceIdType.MESH,
        )
        rdma.start()
        rdma.wait()
        origin = (my - h - 1) % N_DEV
        out_ref[pl.ds(origin * B_PER, B_PER), :, :] = comm_ref[recv_slot]


def _all_gather_x(x):
    return pl.pallas_call(
        _ag_body,
        out_shape=jax.ShapeDtypeStruct((N_DEV * B_PER, SQ, D), jnp.bfloat16),
        in_specs=[pl.BlockSpec(memory_space=pltpu.VMEM)],
        out_specs=pl.BlockSpec(memory_space=pltpu.VMEM),
        scratch_shapes=[
            pltpu.VMEM((2, B_PER, SQ, D), jnp.bfloat16),
            pltpu.SemaphoreType.DMA((2,)),
            pltpu.SemaphoreType.DMA((2,)),
        ],
        compiler_params=pltpu.CompilerParams(collective_id=0),
    )(x)


def _attn_body(x_ref, wq_ref, wo_ref, k_ref, v_ref, part_ref):
    wq = wq_ref[...].astype(jnp.bfloat16)
    wo = wo_ref[...].astype(jnp.bfloat16)
    q = jnp.dot(x_ref[0], wq, preferred_element_type=jnp.float32)

    outs = []
    for h in range(H_PER):
        q_h = q[:, h * DH:(h + 1) * DH].astype(jnp.bfloat16)
        k_h = k_ref[0, :, h * DH:(h + 1) * DH]
        v_h = v_ref[0, :, h * DH:(h + 1) * DH]
        s = lax.dot_general(
            q_h, k_h, (((1,), (1,)), ((), ())),
            preferred_element_type=jnp.float32,
        ) * SCALE
        m = jnp.max(s, axis=1, keepdims=True)
        p = jnp.exp(s - m)
        l = jnp.sum(p, axis=1, keepdims=True)
        pv = jnp.dot(p.astype(jnp.bfloat16), v_h,
                     preferred_element_type=jnp.float32)
        outs.append(pv / l)
    o = jnp.concatenate(outs, axis=1).astype(jnp.bfloat16)
    part_ref[0] = jnp.dot(o, wo, preferred_element_type=jnp.float32)


def _partials(x_all, Wq, Wo, k_loc, v_loc):
    B = N_DEV * B_PER
    return pl.pallas_call(
        _attn_body,
        grid=(B,),
        in_specs=[
            pl.BlockSpec((1, SQ, D), lambda b: (b, 0, 0)),
            pl.BlockSpec((D, H_PER * DH), lambda b: (0, 0)),
            pl.BlockSpec((H_PER * DH, D), lambda b: (0, 0)),
            pl.BlockSpec((1, SKV, H_PER * DH), lambda b: (b, 0, 0)),
            pl.BlockSpec((1, SKV, H_PER * DH), lambda b: (b, 0, 0)),
        ],
        out_specs=pl.BlockSpec((1, SQ, D), lambda b: (b, 0, 0)),
        out_shape=jax.ShapeDtypeStruct((B, SQ, D), jnp.float32),
    )(x_all, Wq, Wo, k_loc, v_loc)


def _rs_body(part_ref, out_ref, sbuf, rbuf, send_sems, recv_sems):
    my = lax.axis_index("i")
    left = (my - 1) % N_DEV
    right = (my + 1) % N_DEV

    barrier_sem = pltpu.get_barrier_semaphore()
    for nbr in [left, right]:
        pl.semaphore_signal(
            barrier_sem, inc=1,
            device_id=(nbr,), device_id_type=pl.DeviceIdType.MESH,
        )
    pl.semaphore_wait(barrier_sem, 2)

    for s in range(N_DEV - 1):
        c = (my - s - 1) % N_DEV
        val = part_ref[pl.ds(c * B_PER, B_PER), :, :]
        if s > 0:
            val = val + rbuf[(s - 1) % 2]
        sbuf[s % 2] = val
        rdma = pltpu.make_async_remote_copy(
            src_ref=sbuf.at[s % 2],
            dst_ref=rbuf.at[s % 2],
            send_sem=send_sems.at[s % 2],
            recv_sem=recv_sems.at[s % 2],
            device_id=(right,),
            device_id_type=pl.DeviceIdType.MESH,
        )
        rdma.start()
        rdma.wait()

    out_ref[...] = part_ref[pl.ds(my * B_PER, B_PER), :, :] + rbuf[(N_DEV - 2) % 2]


def _reduce_scatter(partial):
    return pl.pallas_call(
        _rs_body,
        out_shape=jax.ShapeDtypeStruct((B_PER, SQ, D), jnp.float32),
        in_specs=[pl.BlockSpec(memory_space=pltpu.VMEM)],
        out_specs=pl.BlockSpec(memory_space=pltpu.VMEM),
        scratch_shapes=[
            pltpu.VMEM((2, B_PER, SQ, D), jnp.float32),
            pltpu.VMEM((2, B_PER, SQ, D), jnp.float32),
            pltpu.SemaphoreType.DMA((2,)),
            pltpu.SemaphoreType.DMA((2,)),
        ],
        compiler_params=pltpu.CompilerParams(collective_id=1),
    )(partial)


def kernel(x, Wq, Wo, K_ext, V_ext):
    i = lax.axis_index("i")
    B = N_DEV * B_PER
    k_loc = lax.dynamic_slice_in_dim(K_ext, i * H_PER, H_PER, axis=2)
    v_loc = lax.dynamic_slice_in_dim(V_ext, i * H_PER, H_PER, axis=2)
    k_loc = k_loc.astype(jnp.bfloat16).reshape(B, SKV, H_PER * DH)
    v_loc = v_loc.astype(jnp.bfloat16).reshape(B, SKV, H_PER * DH)
    x_all = _all_gather_x(x)
    partial = _partials(x_all, Wq, Wo, k_loc, v_loc)
    return _reduce_scatter(partial)


# device time: 484825 ns/iter; 1.3321x vs baseline; 1.0864x over previous
import functools

import jax
import jax.numpy as jnp
from jax import lax
from jax.experimental import pallas as pl
from jax.experimental.pallas import tpu as pltpu

N_DEV = 16
B_PER = 2
SQ = 128
SKV = 128
D = 512
H_PER = 8
DH = 64
SCALE = 0.125


def _ag_body(x_ref, out_ref, comm_ref, send_sems, recv_sems):
    my = lax.axis_index("i")
    left = (my - 1) % N_DEV
    right = (my + 1) % N_DEV

    barrier_sem = pltpu.get_barrier_semaphore()
    for nbr in [left, right]:
        pl.semaphore_signal(
            barrier_sem, inc=1,
            device_id=(nbr,), device_id_type=pl.DeviceIdType.MESH,
        )
    pl.semaphore_wait(barrier_sem, 2)

    mine = x_ref[...].astype(jnp.bfloat16)
    out_ref[pl.ds(my * B_PER, B_PER), :, :] = mine
    comm_ref[0] = mine

    for h in range(N_DEV - 1):
        send_slot = h % 2
        recv_slot = (h + 1) % 2
        rdma = pltpu.make_async_remote_copy(
            src_ref=comm_ref.at[send_slot],
            dst_ref=comm_ref.at[recv_slot],
            send_sem=send_sems.at[send_slot],
            recv_sem=recv_sems.at[recv_slot],
            device_id=(right,),
            device_id_type=pl.DeviceIdType.MESH,
        )
        rdma.start()
        rdma.wait()
        origin = (my - h - 1) % N_DEV
        out_ref[pl.ds(origin * B_PER, B_PER), :, :] = comm_ref[recv_slot]


def _all_gather_x(x):
    return pl.pallas_call(
        _ag_body,
        out_shape=jax.ShapeDtypeStruct((N_DEV * B_PER, SQ, D), jnp.bfloat16),
        in_specs=[pl.BlockSpec(memory_space=pltpu.VMEM)],
        out_specs=pl.BlockSpec(memory_space=pltpu.VMEM),
        scratch_shapes=[
            pltpu.VMEM((2, B_PER, SQ, D), jnp.bfloat16),
            pltpu.SemaphoreType.DMA((2,)),
            pltpu.SemaphoreType.DMA((2,)),
        ],
        compiler_params=pltpu.CompilerParams(collective_id=0),
    )(x)


def _attn_body(x_ref, wq_ref, wo_ref, k_ref, v_ref, part_ref):
    wq = wq_ref[...].astype(jnp.bfloat16)
    wo = wo_ref[...].astype(jnp.bfloat16)
    q = jnp.dot(x_ref[0], wq, preferred_element_type=jnp.float32)

    outs = []
    for h in range(H_PER):
        q_h = q[:, h * DH:(h + 1) * DH].astype(jnp.bfloat16)
        k_h = k_ref[0, :, h * DH:(h + 1) * DH]
        v_h = v_ref[0, :, h * DH:(h + 1) * DH]
        s = lax.dot_general(
            q_h, k_h, (((1,), (1,)), ((), ())),
            preferred_element_type=jnp.float32,
        ) * SCALE
        m = jnp.max(s, axis=1, keepdims=True)
        p = jnp.exp(s - m)
        l = jnp.sum(p, axis=1, keepdims=True)
        pv = jnp.dot(p.astype(jnp.bfloat16), v_h,
                     preferred_element_type=jnp.float32)
        outs.append(pv / l)
    o = jnp.concatenate(outs, axis=1).astype(jnp.bfloat16)
    part_ref[0] = jnp.dot(o, wo, preferred_element_type=jnp.float32)


def _partials(x_all, Wq, Wo, k_loc, v_loc):
    B = N_DEV * B_PER
    return pl.pallas_call(
        _attn_body,
        grid=(B,),
        in_specs=[
            pl.BlockSpec((1, SQ, D), lambda b: (b, 0, 0)),
            pl.BlockSpec((D, H_PER * DH), lambda b: (0, 0)),
            pl.BlockSpec((H_PER * DH, D), lambda b: (0, 0)),
            pl.BlockSpec((1, SKV, H_PER * DH), lambda b: (b, 0, 0)),
            pl.BlockSpec((1, SKV, H_PER * DH), lambda b: (b, 0, 0)),
        ],
        out_specs=pl.BlockSpec((1, SQ, D), lambda b: (b, 0, 0)),
        out_shape=jax.ShapeDtypeStruct((B, SQ, D), jnp.float32),
    )(x_all, Wq, Wo, k_loc, v_loc)


def _rs_body(part_ref, out_ref, sbuf, rbuf, send_sems, recv_sems):
    my = lax.axis_index("i")
    left = (my - 1) % N_DEV
    right = (my + 1) % N_DEV

    barrier_sem = pltpu.get_barrier_semaphore()
    for nbr in [left, right]:
        pl.semaphore_signal(
            barrier_sem, inc=1,
            device_id=(nbr,), device_id_type=pl.DeviceIdType.MESH,
        )
    pl.semaphore_wait(barrier_sem, 2)

    for s in range(N_DEV - 1):
        c = (my - s - 1) % N_DEV
        val = part_ref[pl.ds(c * B_PER, B_PER), :, :]
        if s > 0:
            val = val + rbuf[(s - 1) % 2].astype(jnp.float32)
        sbuf[s % 2] = val.astype(jnp.bfloat16)
        rdma = pltpu.make_async_remote_copy(
            src_ref=sbuf.at[s % 2],
            dst_ref=rbuf.at[s % 2],
            send_sem=send_sems.at[s % 2],
            recv_sem=recv_sems.at[s % 2],
            device_id=(right,),
            device_id_type=pl.DeviceIdType.MESH,
        )
        rdma.start()
        rdma.wait()

    out_ref[...] = (part_ref[pl.ds(my * B_PER, B_PER), :, :]
                    + rbuf[(N_DEV - 2) % 2].astype(jnp.float32))


def _reduce_scatter(partial):
    return pl.pallas_call(
        _rs_body,
        out_shape=jax.ShapeDtypeStruct((B_PER, SQ, D), jnp.float32),
        in_specs=[pl.BlockSpec(memory_space=pltpu.VMEM)],
        out_specs=pl.BlockSpec(memory_space=pltpu.VMEM),
        scratch_shapes=[
            pltpu.VMEM((2, B_PER, SQ, D), jnp.bfloat16),
            pltpu.VMEM((2, B_PER, SQ, D), jnp.bfloat16),
            pltpu.SemaphoreType.DMA((2,)),
            pltpu.SemaphoreType.DMA((2,)),
        ],
        compiler_params=pltpu.CompilerParams(collective_id=1),
    )(partial)


def kernel(x, Wq, Wo, K_ext, V_ext):
    i = lax.axis_index("i")
    B = N_DEV * B_PER
    k_loc = lax.dynamic_slice_in_dim(K_ext, i * H_PER, H_PER, axis=2)
    v_loc = lax.dynamic_slice_in_dim(V_ext, i * H_PER, H_PER, axis=2)
    k_loc = k_loc.astype(jnp.bfloat16).reshape(B, SKV, H_PER * DH)
    v_loc = v_loc.astype(jnp.bfloat16).reshape(B, SKV, H_PER * DH)
    x_all = _all_gather_x(x)
    partial = _partials(x_all, Wq, Wo, k_loc, v_loc)
    return _reduce_scatter(partial)


# device time: 236301 ns/iter; 2.7331x vs baseline; 2.0517x over previous
import functools

import jax
import jax.numpy as jnp
from jax import lax
from jax.experimental import pallas as pl
from jax.experimental.pallas import tpu as pltpu

N_DEV = 16
B_PER = 2
SQ = 128
SKV = 128
D = 512
H_PER = 8
DH = 64
SCALE = 0.125


def _ag_body(x_ref, out_ref, comm_ref, send_sems, recv_sems):
    my = lax.axis_index("i")
    left = (my - 1) % N_DEV
    right = (my + 1) % N_DEV

    barrier_sem = pltpu.get_barrier_semaphore()
    for nbr in [left, right]:
        pl.semaphore_signal(
            barrier_sem, inc=1,
            device_id=(nbr,), device_id_type=pl.DeviceIdType.MESH,
        )
    pl.semaphore_wait(barrier_sem, 2)

    mine = x_ref[...].astype(jnp.bfloat16)
    out_ref[pl.ds(my * B_PER, B_PER), :, :] = mine
    comm_ref[0] = mine

    for h in range(N_DEV - 1):
        send_slot = h % 2
        recv_slot = (h + 1) % 2
        rdma = pltpu.make_async_remote_copy(
            src_ref=comm_ref.at[send_slot],
            dst_ref=comm_ref.at[recv_slot],
            send_sem=send_sems.at[send_slot],
            recv_sem=recv_sems.at[recv_slot],
            device_id=(right,),
            device_id_type=pl.DeviceIdType.MESH,
        )
        rdma.start()
        rdma.wait()
        origin = (my - h - 1) % N_DEV
        out_ref[pl.ds(origin * B_PER, B_PER), :, :] = comm_ref[recv_slot]


def _all_gather_x(x):
    return pl.pallas_call(
        _ag_body,
        out_shape=jax.ShapeDtypeStruct((N_DEV * B_PER, SQ, D), jnp.bfloat16),
        in_specs=[pl.BlockSpec(memory_space=pltpu.VMEM)],
        out_specs=pl.BlockSpec(memory_space=pltpu.VMEM),
        scratch_shapes=[
            pltpu.VMEM((2, B_PER, SQ, D), jnp.bfloat16),
            pltpu.SemaphoreType.DMA((2,)),
            pltpu.SemaphoreType.DMA((2,)),
        ],
        compiler_params=pltpu.CompilerParams(collective_id=0),
    )(x)


def _attn_body(x_ref, wq_ref, wo_ref, k_ref, v_ref, part_ref):
    wq = wq_ref[...].astype(jnp.bfloat16)
    wo = wo_ref[...].astype(jnp.bfloat16)
    q = jnp.dot(x_ref[0], wq, preferred_element_type=jnp.float32)

    outs = []
    for h in range(H_PER):
        q_h = q[:, h * DH:(h + 1) * DH].astype(jnp.bfloat16)
        k_h = k_ref[0, :, h * DH:(h + 1) * DH]
        v_h = v_ref[0, :, h * DH:(h + 1) * DH]
        s = lax.dot_general(
            q_h, k_h, (((1,), (1,)), ((), ())),
            preferred_element_type=jnp.float32,
        ) * SCALE
        m = jnp.max(s, axis=1, keepdims=True)
        p = jnp.exp(s - m)
        l = jnp.sum(p, axis=1, keepdims=True)
        pv = jnp.dot(p.astype(jnp.bfloat16), v_h,
                     preferred_element_type=jnp.float32)
        outs.append(pv / l)
    o = jnp.concatenate(outs, axis=1).astype(jnp.bfloat16)
    part_ref[0] = jnp.dot(o, wo, preferred_element_type=jnp.float32)


def _partials(x_all, Wq, Wo, k_loc, v_loc):
    B = N_DEV * B_PER
    return pl.pallas_call(
        _attn_body,
        grid=(B,),
        in_specs=[
            pl.BlockSpec((1, SQ, D), lambda b: (b, 0, 0)),
            pl.BlockSpec((D, H_PER * DH), lambda b: (0, 0)),
            pl.BlockSpec((H_PER * DH, D), lambda b: (0, 0)),
            pl.BlockSpec((1, SKV, H_PER * DH), lambda b: (b, 0, 0)),
            pl.BlockSpec((1, SKV, H_PER * DH), lambda b: (b, 0, 0)),
        ],
        out_specs=pl.BlockSpec((1, SQ, D), lambda b: (b, 0, 0)),
        out_shape=jax.ShapeDtypeStruct((B, SQ, D), jnp.float32),
    )(x_all, Wq, Wo, k_loc, v_loc)


def _rs_body(part_ref, out_ref, sbuf, rbuf, send_sems, recv_sems):
    my = lax.axis_index("i")
    left = (my - 1) % N_DEV
    right = (my + 1) % N_DEV

    barrier_sem = pltpu.get_barrier_semaphore()
    for nbr in [left, right]:
        pl.semaphore_signal(
            barrier_sem, inc=1,
            device_id=(nbr,), device_id_type=pl.DeviceIdType.MESH,
        )
    pl.semaphore_wait(barrier_sem, 2)

    for s in range(N_DEV - 1):
        c = (my - s - 1) % N_DEV
        val = part_ref[pl.ds(c * B_PER, B_PER), :, :]
        if s > 0:
            val = val + rbuf[(s - 1) % 2].astype(jnp.float32)
        sbuf[s % 2] = val.astype(jnp.bfloat16)
        rdma = pltpu.make_async_remote_copy(
            src_ref=sbuf.at[s % 2],
            dst_ref=rbuf.at[s % 2],
            send_sem=send_sems.at[s % 2],
            recv_sem=recv_sems.at[s % 2],
            device_id=(right,),
            device_id_type=pl.DeviceIdType.MESH,
        )
        rdma.start()
        rdma.wait()

    out_ref[...] = (part_ref[pl.ds(my * B_PER, B_PER), :, :]
                    + rbuf[(N_DEV - 2) % 2].astype(jnp.float32))


def _reduce_scatter(partial):
    return pl.pallas_call(
        _rs_body,
        out_shape=jax.ShapeDtypeStruct((B_PER, SQ, D), jnp.float32),
        in_specs=[pl.BlockSpec(memory_space=pltpu.VMEM)],
        out_specs=pl.BlockSpec(memory_space=pltpu.VMEM),
        scratch_shapes=[
            pltpu.VMEM((2, B_PER, SQ, D), jnp.bfloat16),
            pltpu.VMEM((2, B_PER, SQ, D), jnp.bfloat16),
            pltpu.SemaphoreType.DMA((2,)),
            pltpu.SemaphoreType.DMA((2,)),
        ],
        compiler_params=pltpu.CompilerParams(collective_id=1),
    )(partial)


def _fused_body(x_ref, wq_ref, wo_ref, kb_ref, vb_ref, out_ref,
                comm, send_sems, recv_sems):
    my = lax.axis_index("i")
    left = (my - 1) % N_DEV
    right = (my + 1) % N_DEV

    barrier_sem = pltpu.get_barrier_semaphore()
    for nbr in [left, right]:
        pl.semaphore_signal(
            barrier_sem, inc=1,
            device_id=(nbr,), device_id_type=pl.DeviceIdType.MESH,
        )
    pl.semaphore_wait(barrier_sem, 2)

    comm[0, 0] = wq_ref[...].astype(jnp.bfloat16)
    comm[0, 1] = wo_ref[...].astype(jnp.bfloat16)
    out_ref[...] = jnp.zeros((B_PER, SQ, D), jnp.float32)

    xb = [x_ref[b].astype(jnp.bfloat16) for b in range(B_PER)]

    def hop(h, _):
        g = (my - h) % N_DEV
        slot = h % 2

        @pl.when(h < N_DEV - 1)
        def _():
            rdma = pltpu.make_async_remote_copy(
                src_ref=comm.at[slot],
                dst_ref=comm.at[(slot + 1) % 2],
                send_sem=send_sems.at[slot],
                recv_sem=recv_sems.at[(slot + 1) % 2],
                device_id=(right,),
                device_id_type=pl.DeviceIdType.MESH,
            )
            rdma.start()

        wq = comm[slot, 0]
        wo = comm[slot, 1]
        for b in range(B_PER):
            q = jnp.dot(xb[b], wq, preferred_element_type=jnp.float32)
            outs = []
            for hh in range(H_PER):
                q_h = q[:, hh * DH:(hh + 1) * DH].astype(jnp.bfloat16)
                k_h = kb_ref[b, g * H_PER + hh]
                v_h = vb_ref[b, g * H_PER + hh]
                s = lax.dot_general(
                    q_h, k_h, (((1,), (1,)), ((), ())),
                    preferred_element_type=jnp.float32,
                ) * SCALE
                m = jnp.max(s, axis=1, keepdims=True)
                p = jnp.exp(s - m)
                l = jnp.sum(p, axis=1, keepdims=True)
                pv = jnp.dot(p.astype(jnp.bfloat16), v_h,
                             preferred_element_type=jnp.float32)
                outs.append(pv / l)
            o = jnp.concatenate(outs, axis=1).astype(jnp.bfloat16)
            out_ref[b] = out_ref[b] + jnp.dot(
                o, wo, preferred_element_type=jnp.float32)

        @pl.when(h < N_DEV - 1)
        def _():
            rdma = pltpu.make_async_remote_copy(
                src_ref=comm.at[slot],
                dst_ref=comm.at[(slot + 1) % 2],
                send_sem=send_sems.at[slot],
                recv_sem=recv_sems.at[(slot + 1) % 2],
                device_id=(right,),
                device_id_type=pl.DeviceIdType.MESH,
            )
            rdma.wait()

        return 0

    lax.fori_loop(0, N_DEV, hop, 0)


def _fused(x, Wq, Wo, kb, vb):
    return pl.pallas_call(
        _fused_body,
        out_shape=jax.ShapeDtypeStruct((B_PER, SQ, D), jnp.float32),
        in_specs=[pl.BlockSpec(memory_space=pltpu.VMEM)] * 5,
        out_specs=pl.BlockSpec(memory_space=pltpu.VMEM),
        scratch_shapes=[
            pltpu.VMEM((2, 2, D, D), jnp.bfloat16),
            pltpu.SemaphoreType.DMA((2,)),
            pltpu.SemaphoreType.DMA((2,)),
        ],
        compiler_params=pltpu.CompilerParams(collective_id=0),
    )(x, Wq, Wo, kb, vb)


def kernel(x, Wq, Wo, K_ext, V_ext):
    i = lax.axis_index("i")
    kb = lax.dynamic_slice_in_dim(K_ext, i * B_PER, B_PER, axis=0)
    vb = lax.dynamic_slice_in_dim(V_ext, i * B_PER, B_PER, axis=0)
    kb = jnp.transpose(kb.astype(jnp.bfloat16), (0, 2, 1, 3))
    vb = jnp.transpose(vb.astype(jnp.bfloat16), (0, 2, 1, 3))
    return _fused(x, Wq, Wo, kb, vb)


def kernel_v4(x, Wq, Wo, K_ext, V_ext):
    i = lax.axis_index("i")
    B = N_DEV * B_PER
    k_loc = lax.dynamic_slice_in_dim(K_ext, i * H_PER, H_PER, axis=2)
    v_loc = lax.dynamic_slice_in_dim(V_ext, i * H_PER, H_PER, axis=2)
    k_loc = k_loc.astype(jnp.bfloat16).reshape(B, SKV, H_PER * DH)
    v_loc = v_loc.astype(jnp.bfloat16).reshape(B, SKV, H_PER * DH)
    x_all = _all_gather_x(x)
    partial = _partials(x_all, Wq, Wo, k_loc, v_loc)
    return _reduce_scatter(partial)


# device time: 157103 ns/iter; 4.1109x vs baseline; 1.5041x over previous
import functools

import jax
import jax.numpy as jnp
from jax import lax
from jax.experimental import pallas as pl
from jax.experimental.pallas import tpu as pltpu

N_DEV = 16
B_PER = 2
SQ = 128
SKV = 128
D = 512
H_PER = 8
DH = 64
SCALE = 0.125


def _ag_body(x_ref, out_ref, comm_ref, send_sems, recv_sems):
    my = lax.axis_index("i")
    left = (my - 1) % N_DEV
    right = (my + 1) % N_DEV

    barrier_sem = pltpu.get_barrier_semaphore()
    for nbr in [left, right]:
        pl.semaphore_signal(
            barrier_sem, inc=1,
            device_id=(nbr,), device_id_type=pl.DeviceIdType.MESH,
        )
    pl.semaphore_wait(barrier_sem, 2)

    mine = x_ref[...].astype(jnp.bfloat16)
    out_ref[pl.ds(my * B_PER, B_PER), :, :] = mine
    comm_ref[0] = mine

    for h in range(N_DEV - 1):
        send_slot = h % 2
        recv_slot = (h + 1) % 2
        rdma = pltpu.make_async_remote_copy(
            src_ref=comm_ref.at[send_slot],
            dst_ref=comm_ref.at[recv_slot],
            send_sem=send_sems.at[send_slot],
            recv_sem=recv_sems.at[recv_slot],
            device_id=(right,),
            device_id_type=pl.DeviceIdType.MESH,
        )
        rdma.start()
        rdma.wait()
        origin = (my - h - 1) % N_DEV
        out_ref[pl.ds(origin * B_PER, B_PER), :, :] = comm_ref[recv_slot]


def _all_gather_x(x):
    return pl.pallas_call(
        _ag_body,
        out_shape=jax.ShapeDtypeStruct((N_DEV * B_PER, SQ, D), jnp.bfloat16),
        in_specs=[pl.BlockSpec(memory_space=pltpu.VMEM)],
        out_specs=pl.BlockSpec(memory_space=pltpu.VMEM),
        scratch_shapes=[
            pltpu.VMEM((2, B_PER, SQ, D), jnp.bfloat16),
            pltpu.SemaphoreType.DMA((2,)),
            pltpu.SemaphoreType.DMA((2,)),
        ],
        compiler_params=pltpu.CompilerParams(collective_id=0),
    )(x)


def _attn_body(x_ref, wq_ref, wo_ref, k_ref, v_ref, part_ref):
    wq = wq_ref[...].astype(jnp.bfloat16)
    wo = wo_ref[...].astype(jnp.bfloat16)
    q = jnp.dot(x_ref[0], wq, preferred_element_type=jnp.float32)

    outs = []
    for h in range(H_PER):
        q_h = q[:, h * DH:(h + 1) * DH].astype(jnp.bfloat16)
        k_h = k_ref[0, :, h * DH:(h + 1) * DH]
        v_h = v_ref[0, :, h * DH:(h + 1) * DH]
        s = lax.dot_general(
            q_h, k_h, (((1,), (1,)), ((), ())),
            preferred_element_type=jnp.float32,
        ) * SCALE
        m = jnp.max(s, axis=1, keepdims=True)
        p = jnp.exp(s - m)
        l = jnp.sum(p, axis=1, keepdims=True)
        pv = jnp.dot(p.astype(jnp.bfloat16), v_h,
                     preferred_element_type=jnp.float32)
        outs.append(pv / l)
    o = jnp.concatenate(outs, axis=1).astype(jnp.bfloat16)
    part_ref[0] = jnp.dot(o, wo, preferred_element_type=jnp.float32)


def _partials(x_all, Wq, Wo, k_loc, v_loc):
    B = N_DEV * B_PER
    return pl.pallas_call(
        _attn_body,
        grid=(B,),
        in_specs=[
            pl.BlockSpec((1, SQ, D), lambda b: (b, 0, 0)),
            pl.BlockSpec((D, H_PER * DH), lambda b: (0, 0)),
            pl.BlockSpec((H_PER * DH, D), lambda b: (0, 0)),
            pl.BlockSpec((1, SKV, H_PER * DH), lambda b: (b, 0, 0)),
            pl.BlockSpec((1, SKV, H_PER * DH), lambda b: (b, 0, 0)),
        ],
        out_specs=pl.BlockSpec((1, SQ, D), lambda b: (b, 0, 0)),
        out_shape=jax.ShapeDtypeStruct((B, SQ, D), jnp.float32),
    )(x_all, Wq, Wo, k_loc, v_loc)


def _rs_body(part_ref, out_ref, sbuf, rbuf, send_sems, recv_sems):
    my = lax.axis_index("i")
    left = (my - 1) % N_DEV
    right = (my + 1) % N_DEV

    barrier_sem = pltpu.get_barrier_semaphore()
    for nbr in [left, right]:
        pl.semaphore_signal(
            barrier_sem, inc=1,
            device_id=(nbr,), device_id_type=pl.DeviceIdType.MESH,
        )
    pl.semaphore_wait(barrier_sem, 2)

    for s in range(N_DEV - 1):
        c = (my - s - 1) % N_DEV
        val = part_ref[pl.ds(c * B_PER, B_PER), :, :]
        if s > 0:
            val = val + rbuf[(s - 1) % 2].astype(jnp.float32)
        sbuf[s % 2] = val.astype(jnp.bfloat16)
        rdma = pltpu.make_async_remote_copy(
            src_ref=sbuf.at[s % 2],
            dst_ref=rbuf.at[s % 2],
            send_sem=send_sems.at[s % 2],
            recv_sem=recv_sems.at[s % 2],
            device_id=(right,),
            device_id_type=pl.DeviceIdType.MESH,
        )
        rdma.start()
        rdma.wait()

    out_ref[...] = (part_ref[pl.ds(my * B_PER, B_PER), :, :]
                    + rbuf[(N_DEV - 2) % 2].astype(jnp.float32))


def _reduce_scatter(partial):
    return pl.pallas_call(
        _rs_body,
        out_shape=jax.ShapeDtypeStruct((B_PER, SQ, D), jnp.float32),
        in_specs=[pl.BlockSpec(memory_space=pltpu.VMEM)],
        out_specs=pl.BlockSpec(memory_space=pltpu.VMEM),
        scratch_shapes=[
            pltpu.VMEM((2, B_PER, SQ, D), jnp.bfloat16),
            pltpu.VMEM((2, B_PER, SQ, D), jnp.bfloat16),
            pltpu.SemaphoreType.DMA((2,)),
            pltpu.SemaphoreType.DMA((2,)),
        ],
        compiler_params=pltpu.CompilerParams(collective_id=1),
    )(partial)


def _fused_body(x_ref, wq_ref, wo_ref, kb_ref, vb_ref, out_ref,
                comm, send_sems, recv_sems):
    my = lax.axis_index("i")
    left = (my - 1) % N_DEV
    right = (my + 1) % N_DEV

    barrier_sem = pltpu.get_barrier_semaphore()
    for nbr in [left, right]:
        pl.semaphore_signal(
            barrier_sem, inc=1,
            device_id=(nbr,), device_id_type=pl.DeviceIdType.MESH,
        )
    pl.semaphore_wait(barrier_sem, 2)

    comm[0, 0] = wq_ref[...].astype(jnp.bfloat16)
    comm[0, 1] = wo_ref[...].astype(jnp.bfloat16)
    out_ref[...] = jnp.zeros((B_PER, SQ, D), jnp.float32)

    xb = [x_ref[b].astype(jnp.bfloat16) for b in range(B_PER)]

    def hop(h, _):
        g = (my - h) % N_DEV
        slot = h % 2

        @pl.when(h < N_DEV - 1)
        def _():
            rdma = pltpu.make_async_remote_copy(
                src_ref=comm.at[slot],
                dst_ref=comm.at[(slot + 1) % 2],
                send_sem=send_sems.at[slot],
                recv_sem=recv_sems.at[(slot + 1) % 2],
                device_id=(right,),
                device_id_type=pl.DeviceIdType.MESH,
            )
            rdma.start()

        wq = comm[slot, 0]
        wo = comm[slot, 1]
        for b in range(B_PER):
            q = jnp.dot(xb[b], wq, preferred_element_type=jnp.float32)
            outs = []
            for hh in range(H_PER):
                q_h = q[:, hh * DH:(hh + 1) * DH].astype(jnp.bfloat16)
                k_h = kb_ref[b, g * H_PER + hh]
                v_h = vb_ref[b, g * H_PER + hh]
                s = lax.dot_general(
                    q_h, k_h, (((1,), (1,)), ((), ())),
                    preferred_element_type=jnp.float32,
                ) * SCALE
                m = jnp.max(s, axis=1, keepdims=True)
                p = jnp.exp(s - m)
                l = jnp.sum(p, axis=1, keepdims=True)
                pv = jnp.dot(p.astype(jnp.bfloat16), v_h,
                             preferred_element_type=jnp.float32)
                outs.append(pv / l)
            o = jnp.concatenate(outs, axis=1).astype(jnp.bfloat16)
            out_ref[b] = out_ref[b] + jnp.dot(
                o, wo, preferred_element_type=jnp.float32)

        @pl.when(h < N_DEV - 1)
        def _():
            rdma = pltpu.make_async_remote_copy(
                src_ref=comm.at[slot],
                dst_ref=comm.at[(slot + 1) % 2],
                send_sem=send_sems.at[slot],
                recv_sem=recv_sems.at[(slot + 1) % 2],
                device_id=(right,),
                device_id_type=pl.DeviceIdType.MESH,
            )
            rdma.wait()

        return 0

    lax.fori_loop(0, N_DEV, hop, 0)


def _fused(x, Wq, Wo, kb, vb):
    return pl.pallas_call(
        _fused_body,
        out_shape=jax.ShapeDtypeStruct((B_PER, SQ, D), jnp.float32),
        in_specs=[pl.BlockSpec(memory_space=pltpu.VMEM)] * 5,
        out_specs=pl.BlockSpec(memory_space=pltpu.VMEM),
        scratch_shapes=[
            pltpu.VMEM((2, 2, D, D), jnp.bfloat16),
            pltpu.SemaphoreType.DMA((2,)),
            pltpu.SemaphoreType.DMA((2,)),
        ],
        compiler_params=pltpu.CompilerParams(collective_id=0),
    )(x, Wq, Wo, kb, vb)


def _fused2_body(x_ref, wq_ref, wo_ref, kb_ref, vb_ref, out_ref,
                 commR, commL, sendR, recvR, sendL, recvL):
    my = lax.axis_index("i")
    left = (my - 1) % N_DEV
    right = (my + 1) % N_DEV

    barrier_sem = pltpu.get_barrier_semaphore()
    for nbr in [left, right]:
        pl.semaphore_signal(
            barrier_sem, inc=1,
            device_id=(nbr,), device_id_type=pl.DeviceIdType.MESH,
        )
    pl.semaphore_wait(barrier_sem, 2)

    mine_q = wq_ref[...].astype(jnp.bfloat16)
    mine_o = wo_ref[...].astype(jnp.bfloat16)
    commR[0, 0] = mine_q
    commR[0, 1] = mine_o
    commL[0, 0] = mine_q
    commL[0, 1] = mine_o
    out_ref[...] = jnp.zeros((B_PER, SQ, D), jnp.float32)

    xb = [x_ref[b].astype(jnp.bfloat16) for b in range(B_PER)]

    def add_group(wq, wo, g):
        for b in range(B_PER):
            q = jnp.dot(xb[b], wq, preferred_element_type=jnp.float32)
            outs = []
            for hh in range(H_PER):
                q_h = q[:, hh * DH:(hh + 1) * DH].astype(jnp.bfloat16)
                k_h = kb_ref[b, g * H_PER + hh]
                v_h = vb_ref[b, g * H_PER + hh]
                s = lax.dot_general(
                    q_h, k_h, (((1,), (1,)), ((), ())),
                    preferred_element_type=jnp.float32,
                ) * SCALE
                m = jnp.max(s, axis=1, keepdims=True)
                p = jnp.exp(s - m)
                l = jnp.sum(p, axis=1, keepdims=True)
                pv = jnp.dot(p.astype(jnp.bfloat16), v_h,
                             preferred_element_type=jnp.float32)
                outs.append(pv / l)
            o = jnp.concatenate(outs, axis=1).astype(jnp.bfloat16)
            out_ref[b] = out_ref[b] + jnp.dot(
                o, wo, preferred_element_type=jnp.float32)

    def hop(h, _):
        slot = h % 2
        nslot = (slot + 1) % 2

        rdmaR = pltpu.make_async_remote_copy(
            src_ref=commR.at[slot], dst_ref=commR.at[nslot],
            send_sem=sendR.at[slot], recv_sem=recvR.at[nslot],
            device_id=(right,), device_id_type=pl.DeviceIdType.MESH,
        )
        rdmaR.start()

        @pl.when(h < 7)
        def _():
            rdmaL = pltpu.make_async_remote_copy(
                src_ref=commL.at[slot], dst_ref=commL.at[nslot],
                send_sem=sendL.at[slot], recv_sem=recvL.at[nslot],
                device_id=(left,), device_id_type=pl.DeviceIdType.MESH,
            )
            rdmaL.start()

        add_group(commR[slot, 0], commR[slot, 1], (my - h) % N_DEV)

        @pl.when(h >= 1)
        def _():
            add_group(commL[slot, 0], commL[slot, 1], (my + h) % N_DEV)

        rdmaR2 = pltpu.make_async_remote_copy(
            src_ref=commR.at[slot], dst_ref=commR.at[nslot],
            send_sem=sendR.at[slot], recv_sem=recvR.at[nslot],
            device_id=(right,), device_id_type=pl.DeviceIdType.MESH,
        )
        rdmaR2.wait()

        @pl.when(h < 7)
        def _():
            rdmaL2 = pltpu.make_async_remote_copy(
                src_ref=commL.at[slot], dst_ref=commL.at[nslot],
                send_sem=sendL.at[slot], recv_sem=recvL.at[nslot],
                device_id=(left,), device_id_type=pl.DeviceIdType.MESH,
            )
            rdmaL2.wait()

        return 0

    lax.fori_loop(0, 8, hop, 0)
    add_group(commR[0, 0], commR[0, 1], (my - 8) % N_DEV)


def _fused2(x, Wq, Wo, kb, vb):
    return pl.pallas_call(
        _fused2_body,
        out_shape=jax.ShapeDtypeStruct((B_PER, SQ, D), jnp.float32),
        in_specs=[pl.BlockSpec(memory_space=pltpu.VMEM)] * 5,
        out_specs=pl.BlockSpec(memory_space=pltpu.VMEM),
        scratch_shapes=[
            pltpu.VMEM((2, 2, D, D), jnp.bfloat16),
            pltpu.VMEM((2, 2, D, D), jnp.bfloat16),
            pltpu.SemaphoreType.DMA((2,)),
            pltpu.SemaphoreType.DMA((2,)),
            pltpu.SemaphoreType.DMA((2,)),
            pltpu.SemaphoreType.DMA((2,)),
        ],
        compiler_params=pltpu.CompilerParams(collective_id=0),
    )(x, Wq, Wo, kb, vb)


def kernel(x, Wq, Wo, K_ext, V_ext):
    i = lax.axis_index("i")
    kb = lax.dynamic_slice_in_dim(K_ext, i * B_PER, B_PER, axis=0)
    vb = lax.dynamic_slice_in_dim(V_ext, i * B_PER, B_PER, axis=0)
    kb = jnp.transpose(kb.astype(jnp.bfloat16), (0, 2, 1, 3))
    vb = jnp.transpose(vb.astype(jnp.bfloat16), (0, 2, 1, 3))
    return _fused2(x, Wq, Wo, kb, vb)


def kernel_v4(x, Wq, Wo, K_ext, V_ext):
    i = lax.axis_index("i")
    B = N_DEV * B_PER
    k_loc = lax.dynamic_slice_in_dim(K_ext, i * H_PER, H_PER, axis=2)
    v_loc = lax.dynamic_slice_in_dim(V_ext, i * H_PER, H_PER, axis=2)
    k_loc = k_loc.astype(jnp.bfloat16).reshape(B, SKV, H_PER * DH)
    v_loc = v_loc.astype(jnp.bfloat16).reshape(B, SKV, H_PER * DH)
    x_all = _all_gather_x(x)
    partial = _partials(x_all, Wq, Wo, k_loc, v_loc)
    return _reduce_scatter(partial)
